# Initial kernel scaffold; baseline (speedup 1.0000x reference)
import functools

import jax
import jax.numpy as jnp
from jax import lax
from jax.experimental import pallas as pl
from jax.experimental.pallas import tpu as pltpu
from jax.experimental.pallas import tpu_sc as plsc

LR = 0.01
EPS = 1e-10
M = 1000000
D = 32
B = 16384

NS = 16
C = B // NS
SB = 128
NB = C // SB
L = 16


def _rsqrt(s):
    s = jnp.maximum(s, 1e-37)
    i = plsc.bitcast(s, jnp.int32)
    y = plsc.bitcast(jnp.int32(0x5F3759DF) - (i >> 1), jnp.float32)
    for _ in range(3):
        y = y * (1.5 - 0.5 * s * y * y)
    return s, y


def _sc_body(idx_hbm, grad_hbm, jot_hbm, emb_ref, state_ref,
             slot_tab, gsum, cntt,
             idx_v, jv, rep_v, gbuf, obuf, emb_g, state_g, sem):
    tid = lax.axis_index("s")
    base = tid * C

    zeros16 = jnp.zeros((L,), jnp.float32)
    ones16 = jnp.ones((L,), jnp.float32)

    pltpu.sync_copy(idx_hbm.at[pl.ds(tid * NB, NB)], idx_v)
    pltpu.sync_copy(jot_hbm.at[pl.ds(tid * NB, NB)], jv)
    for j in range(NB):
        pltpu.sync_copy(jv.at[j], slot_tab.at[idx_v.at[j]])

    def _zrow(r, carry):
        gbuf[r, pl.ds(0, L)] = zeros16
        gbuf[r, pl.ds(L, L)] = zeros16
        obuf[r] = zeros16
        return carry
    lax.fori_loop(0, C, _zrow, 0)
    pltpu.sync_copy(gbuf, gsum.at[pl.ds(base, C)])
    pltpu.sync_copy(obuf, cntt.at[pl.ds(base, C)])

    plsc.subcore_barrier()

    descs = [pltpu.async_copy(slot_tab.at[idx_v.at[j]], rep_v.at[j], sem)
             for j in range(NB)]
    for d in descs:
        d.wait()
    pltpu.sync_copy(grad_hbm.at[pl.ds(base, C)], gbuf)

    def _orow(r, carry):
        obuf[r] = ones16
        return carry
    lax.fori_loop(0, C, _orow, 0)

    for j in range(NB):
        pltpu.sync_copy(gbuf.at[pl.ds(j * SB, SB)], gsum.at[rep_v.at[j]],
                        add=True)
        pltpu.sync_copy(obuf.at[pl.ds(j * SB, SB)], cntt.at[rep_v.at[j]],
                        add=True)

    plsc.subcore_barrier()

    descs = [pltpu.async_copy(gsum.at[rep_v.at[j]],
                              gbuf.at[pl.ds(j * SB, SB)], sem)
             for j in range(NB)]
    descs += [pltpu.async_copy(cntt.at[rep_v.at[j]],
                               obuf.at[pl.ds(j * SB, SB)], sem)
              for j in range(NB)]
    descs += [pltpu.async_copy(emb_ref.at[idx_v.at[j]],
                               emb_g.at[pl.ds(j * SB, SB)], sem)
              for j in range(NB)]
    descs += [pltpu.async_copy(state_ref.at[idx_v.at[j]],
                               state_g.at[pl.ds(j * SB, SB)], sem)
              for j in range(NB)]
    for d in descs:
        d.wait()

    def _row(r, carry):
        cnt = obuf[r]
        m0 = gbuf[r, pl.ds(0, L)] / cnt
        m1 = gbuf[r, pl.ds(L, L)] / cnt
        s0 = state_g[r, pl.ds(0, L)] + m0 * m0
        s1 = state_g[r, pl.ds(L, L)] + m1 * m1
        state_g[r, pl.ds(0, L)] = s0
        state_g[r, pl.ds(L, L)] = s1
        sc0, y0 = _rsqrt(s0)
        sc1, y1 = _rsqrt(s1)
        d0 = sc0 * y0 + EPS
        d1 = sc1 * y1 + EPS
        emb_g[r, pl.ds(0, L)] = emb_g[r, pl.ds(0, L)] - LR * m0 / d0
        emb_g[r, pl.ds(L, L)] = emb_g[r, pl.ds(L, L)] - LR * m1 / d1
        return carry
    lax.fori_loop(0, C, _row, 0)

    plsc.subcore_barrier()

    descs = [pltpu.async_copy(state_g.at[pl.ds(j * SB, SB)],
                              state_ref.at[idx_v.at[j]], sem)
             for j in range(NB)]
    descs += [pltpu.async_copy(emb_g.at[pl.ds(j * SB, SB)],
                               emb_ref.at[idx_v.at[j]], sem)
              for j in range(NB)]
    for d in descs:
        d.wait()


_sc_update = functools.partial(
    pl.kernel,
    out_type=(),
    mesh=plsc.VectorSubcoreMesh(
        core_axis_name="c", subcore_axis_name="s", num_cores=1),
    scratch_types=[
        pltpu.VMEM_SHARED((M,), jnp.int32),
        pltpu.VMEM_SHARED((B, D), jnp.float32),
        pltpu.VMEM_SHARED((B, L), jnp.float32),
        pltpu.VMEM((NB, SB), jnp.int32),
        pltpu.VMEM((NB, SB), jnp.int32),
        pltpu.VMEM((NB, SB), jnp.int32),
        pltpu.VMEM((C, D), jnp.float32),
        pltpu.VMEM((C, L), jnp.float32),
        pltpu.VMEM((C, D), jnp.float32),
        pltpu.VMEM((C, D), jnp.float32),
        pltpu.SemaphoreType.DMA,
    ],
)(_sc_body)


def kernel(idx, grad, emb, state):
    idx2 = idx.reshape(B // SB, SB)
    jot = jnp.arange(B, dtype=jnp.int32).reshape(B // SB, SB)
    emb_ref = jax.new_ref(emb)
    state_ref = jax.new_ref(state)
    _sc_update(idx2, grad, jot, emb_ref, state_ref)
    return jax.freeze(emb_ref), jax.freeze(state_ref)


# speedup vs baseline: 1.0836x; 1.0836x over previous
import functools

import jax
import jax.numpy as jnp
from jax import lax
from jax.experimental import pallas as pl
from jax.experimental.pallas import tpu as pltpu
from jax.experimental.pallas import tpu_sc as plsc

LR = 0.01
EPS = 1e-10
M = 1000000
D = 32
B = 16384

NS = 16
C = B // NS
SB = 128
NB = C // SB
L = 16


def _rsqrt(s):
    s = jnp.maximum(s, 1e-37)
    i = lax.bitcast_convert_type(s, jnp.int32)
    y = lax.bitcast_convert_type(jnp.int32(0x5F3759DF) - (i >> 1),
                                 jnp.float32)
    for _ in range(3):
        y = y * (1.5 - 0.5 * s * y * y)
    return s, y


def _sc_body(idx_hbm, grad_hbm, emb_hbm, state_hbm, emb_ref, state_ref,
             slot_tab, gsum, cntt,
             idx_v, rep_v, gbuf, obuf, rowbuf, sem):
    tid = lax.axis_index("s")
    base = tid * C

    zeros16 = jnp.zeros((L,), jnp.float32)
    ones16 = jnp.ones((L,), jnp.float32)
    iota16 = lax.iota(jnp.int32, L)

    pltpu.sync_copy(idx_hbm.at[pl.ds(tid * NB, NB)], idx_v)
    for j in range(NB):
        def _jfill(k, carry, j=j):
            rep_v[j, pl.ds(k * L, L)] = iota16 + (base + j * SB + k * L)
            return carry
        lax.fori_loop(0, SB // L, _jfill, 0)
        pltpu.sync_copy(rep_v.at[j], slot_tab.at[idx_v.at[j]])

    def _zrow(r, carry):
        gbuf[r, pl.ds(0, L)] = zeros16
        gbuf[r, pl.ds(L, L)] = zeros16
        obuf[r] = zeros16
        return carry
    lax.fori_loop(0, SB, _zrow, 0)
    for j in range(NB):
        pltpu.sync_copy(gbuf, gsum.at[pl.ds(base + j * SB, SB)])
        pltpu.sync_copy(obuf, cntt.at[pl.ds(base + j * SB, SB)])

    plsc.subcore_barrier()

    descs = [pltpu.async_copy(slot_tab.at[idx_v.at[j]], rep_v.at[j], sem)
             for j in range(NB)]
    for d in descs:
        d.wait()

    def _orow(r, carry):
        obuf[r] = ones16
        return carry
    lax.fori_loop(0, SB, _orow, 0)

    for j in range(NB):
        pltpu.sync_copy(grad_hbm.at[pl.ds(base + j * SB, SB)], gbuf)
        pltpu.sync_copy(gbuf, gsum.at[rep_v.at[j]], add=True)
        pltpu.sync_copy(obuf, cntt.at[rep_v.at[j]], add=True)

    plsc.subcore_barrier()

    for j in range(NB):
        pltpu.sync_copy(gsum.at[rep_v.at[j]], gbuf)
        pltpu.sync_copy(cntt.at[rep_v.at[j]], obuf)
        pltpu.sync_copy(state_hbm.at[idx_v.at[j]], rowbuf)

        def _srow(r, carry):
            cnt = obuf[r]
            m0 = gbuf[r, pl.ds(0, L)] / cnt
            m1 = gbuf[r, pl.ds(L, L)] / cnt
            s0 = rowbuf[r, pl.ds(0, L)] + m0 * m0
            s1 = rowbuf[r, pl.ds(L, L)] + m1 * m1
            rowbuf[r, pl.ds(0, L)] = s0
            rowbuf[r, pl.ds(L, L)] = s1
            sc0, y0 = _rsqrt(s0)
            sc1, y1 = _rsqrt(s1)
            gbuf[r, pl.ds(0, L)] = LR * m0 / (sc0 * y0 + EPS)
            gbuf[r, pl.ds(L, L)] = LR * m1 / (sc1 * y1 + EPS)
            return carry
        lax.fori_loop(0, SB, _srow, 0)
        pltpu.sync_copy(rowbuf, state_ref.at[idx_v.at[j]])

        pltpu.sync_copy(emb_hbm.at[idx_v.at[j]], rowbuf)

        def _erow(r, carry):
            rowbuf[r, pl.ds(0, L)] = (rowbuf[r, pl.ds(0, L)]
                                      - gbuf[r, pl.ds(0, L)])
            rowbuf[r, pl.ds(L, L)] = (rowbuf[r, pl.ds(L, L)]
                                      - gbuf[r, pl.ds(L, L)])
            return carry
        lax.fori_loop(0, SB, _erow, 0)
        pltpu.sync_copy(rowbuf, emb_ref.at[idx_v.at[j]])


@functools.lru_cache(maxsize=1)
def _make_sc_update():
  return pl.kernel(
    _sc_body,
    out_type=(),
    mesh=plsc.VectorSubcoreMesh(
        core_axis_name="c", subcore_axis_name="s",
        num_cores=1, num_subcores=NS),
    scratch_types=[
        pltpu.VMEM_SHARED((M,), jnp.int32),
        pltpu.VMEM_SHARED((B, D), jnp.float32),
        pltpu.VMEM_SHARED((B, L), jnp.float32),
        pltpu.VMEM((NB, SB), jnp.int32),
        pltpu.VMEM((NB, SB), jnp.int32),
        pltpu.VMEM((SB, D), jnp.float32),
        pltpu.VMEM((SB, L), jnp.float32),
        pltpu.VMEM((SB, D), jnp.float32),
        pltpu.SemaphoreType.DMA,
    ],
    compiler_params=pltpu.CompilerParams(use_tc_tiling_on_sc=False),
  )


def kernel(idx, grad, emb, state):
    idx2 = idx.reshape(B // SB, SB)
    emb_ref = jax.new_ref(emb)
    state_ref = jax.new_ref(state)
    _make_sc_update()(idx2, grad, emb, state, emb_ref, state_ref)
    return jax.freeze(emb_ref), jax.freeze(state_ref)
